# Initial kernel scaffold; baseline (speedup 1.0000x reference)
import jax
import jax.numpy as jnp
from jax.experimental import pallas as pl


def kernel(input_ids, states, E, Wx, Wh, b, Wout, bout, mask):
    raise NotImplementedError("write your pallas kernel here")


# speedup vs baseline: 3.6018x; 3.6018x over previous
import jax
import jax.numpy as jnp
from jax.experimental import pallas as pl

_B, _S, _V, _D_EMB, _H = 64, 60, 128, 256, 1024


def _onestep_kernel(ids_ref, h0_ref, E_ref, Wx_ref, Wh_ref, b_ref, Wout_ref,
                    bout_ref, mask_ref, noise_ref,
                    ids_out_ref, h_out_ref, logits_out_ref):
    EWxb = jnp.dot(E_ref[...], Wx_ref[...],
                   preferred_element_type=jnp.float32) + b_ref[...]
    Wh = Wh_ref[...]
    iota_v = jax.lax.broadcasted_iota(jnp.int32, (_V, _B), 0)

    def step(t, h):
        tok = ids_ref[pl.ds(t, 1), :]
        onehot_vb = (iota_v == tok).astype(jnp.float32)
        gx = jax.lax.dot_general(onehot_vb, EWxb, (((0,), (0,)), ((), ())),
                                 preferred_element_type=jnp.float32)
        gh = jnp.dot(h, Wh, preferred_element_type=jnp.float32)
        z = jax.nn.sigmoid(gx[:, :_H] + gh[:, :_H])
        r = jax.nn.sigmoid(gx[:, _H:2 * _H] + gh[:, _H:2 * _H])
        hh = jnp.tanh(gx[:, 2 * _H:] + r * gh[:, 2 * _H:])
        return z * h + (1.0 - z) * hh

    h = jax.lax.fori_loop(0, _S, step, h0_ref[...])
    h_out_ref[...] = h
    logits = (jnp.dot(h, Wout_ref[...], preferred_element_type=jnp.float32)
              + bout_ref[...] + mask_ref[...])
    logits_out_ref[...] = logits
    sample = jnp.argmax(logits + noise_ref[...], axis=1).astype(jnp.int32)
    ids_out_ref[...] = sample[:, None]


def kernel(input_ids, states, E, Wx, Wh, b, Wout, bout, mask):
    ids = input_ids.astype(jnp.int32).T
    noise = jax.random.gumbel(jax.random.key(1), (_B, _V), jnp.float32)
    out_types = (
        jax.ShapeDtypeStruct((_B, 1), jnp.int32),
        jax.ShapeDtypeStruct((_B, _H), jnp.float32),
        jax.ShapeDtypeStruct((_B, _V), jnp.float32),
    )
    ids_out, h_out, logits = pl.pallas_call(
        _onestep_kernel,
        out_shape=out_types,
    )(ids, states, E, Wx, Wh, b.reshape(1, 3 * _H), Wout,
      bout.reshape(1, _V), mask.reshape(1, _V), noise)
    return ids_out.reshape(_B), h_out, logits
